# Initial kernel scaffold; baseline (speedup 1.0000x reference)
import jax
import jax.numpy as jnp
from jax.experimental import pallas as pl


def kernel(features, neigh_idx, W_pool0, b_pool0, W_fc0, b_fc0, bn_gamma, bn_beta, W_pool1, b_pool1, W_fc1, b_fc1):
    raise NotImplementedError("write your pallas kernel here")


# speedup vs baseline: 2.4340x; 2.4340x over previous
import functools

import jax
import jax.numpy as jnp
from jax import lax
from jax.experimental import pallas as pl
from jax.experimental.pallas import tpu as pltpu
from jax.experimental.pallas import tpu_sc as plsc

N = 50000
D = 256
S = 5
L = 16

NC = 2
NS = 16
NW = NC * NS

NP = 50176
NPW = NP // NW
CHUNK = 56
NCHUNK = NPW // CHUNK

BM = 2000
GRID = N // BM


def _sc_gather_max_body(table_hbm, idx_hbm, out_hbm, idx_v, rows_v, out_v, sem):
    wid = lax.axis_index("s") * NC + lax.axis_index("c")
    node_base = wid * NPW

    def chunk_body(ci, carry):
        base = node_base + ci * CHUNK
        pltpu.sync_copy(idx_hbm.at[pl.ds(base * S, CHUNK * S)], idx_v)
        pltpu.async_copy(table_hbm.at[idx_v], rows_v, sem).wait()

        def node_body(j, carry2):
            r = j * S
            for k in range(D // L):
                sl = pl.ds(k * L, L)
                m = rows_v[r, sl]
                for s in range(1, S):
                    m = jnp.maximum(m, rows_v[r + s, sl])
                out_v[j, sl] = m
            return carry2

        lax.fori_loop(0, CHUNK, node_body, 0, unroll=False)
        pltpu.sync_copy(out_v, out_hbm.at[pl.ds(base, CHUNK)])
        return carry

    lax.fori_loop(0, NCHUNK, chunk_body, 0, unroll=False)


def _sc_gather_max(table, idx_flat):
    mesh = plsc.VectorSubcoreMesh(core_axis_name="c", subcore_axis_name="s")
    return pl.kernel(
        _sc_gather_max_body,
        out_type=jax.ShapeDtypeStruct((NP, D), jnp.float32),
        mesh=mesh,
        scratch_types=[
            pltpu.VMEM((CHUNK * S,), jnp.int32),
            pltpu.VMEM((CHUNK * S, D), jnp.float32),
            pltpu.VMEM((CHUNK, D), jnp.float32),
            pltpu.SemaphoreType.DMA,
        ],
        name="sc_gather_max",
    )(table, idx_flat)


def _mm_relu_body(x_ref, w_ref, b_ref, o_ref):
    acc = jnp.dot(x_ref[...], w_ref[...], preferred_element_type=jnp.float32)
    o_ref[...] = jnp.maximum(acc + b_ref[...], 0.0)


def _fc0_body(x_ref, agg_ref, wa_ref, wb_ref, b_ref, z_ref, s_ref, ss_ref):
    z = jnp.dot(x_ref[...], wa_ref[...], preferred_element_type=jnp.float32)
    z += jnp.dot(agg_ref[...], wb_ref[...], preferred_element_type=jnp.float32)
    z = jnp.maximum(z + b_ref[...], 0.0)
    z_ref[...] = z

    @pl.when(pl.program_id(0) == 0)
    def _():
        s_ref[...] = jnp.zeros_like(s_ref)
        ss_ref[...] = jnp.zeros_like(ss_ref)

    s_ref[...] += jnp.sum(z, axis=0, keepdims=True)
    ss_ref[...] += jnp.sum(z * z, axis=0, keepdims=True)


def _bn_body(z_ref, s_ref, ss_ref, g_ref, be_ref, wp_ref, bp_ref,
             on_ref, h1_ref):
    mean = s_ref[...] / N
    var = ss_ref[...] / N - mean * mean
    y = (z_ref[...] - mean) / jnp.sqrt(var + 1e-5) * g_ref[...] + be_ref[...]
    nrm = jnp.sqrt(jnp.sum(y * y, axis=1, keepdims=True)) + 1e-6
    on = y / nrm
    on_ref[...] = on
    acc = jnp.dot(on, wp_ref[...], preferred_element_type=jnp.float32)
    h1_ref[...] = jnp.maximum(acc + bp_ref[...], 0.0)


def _fc1_body(x_ref, agg_ref, wa_ref, wb_ref, b_ref, o_ref):
    o = jnp.dot(x_ref[...], wa_ref[...], preferred_element_type=jnp.float32)
    o += jnp.dot(agg_ref[...], wb_ref[...], preferred_element_type=jnp.float32)
    o_ref[...] = o + b_ref[...]


def _row_spec():
    return pl.BlockSpec((BM, D), lambda i: (i, 0))


def _full_spec(shape):
    return pl.BlockSpec(shape, lambda i: tuple(0 for _ in shape))


def _mm_relu(x, w, b):
    return pl.pallas_call(
        _mm_relu_body,
        grid=(GRID,),
        in_specs=[_row_spec(), _full_spec((D, D)), _full_spec((1, D))],
        out_specs=_row_spec(),
        out_shape=jax.ShapeDtypeStruct((N, D), jnp.float32),
    )(x, w, b)


def _fc0(x, agg, wa, wb, b):
    return pl.pallas_call(
        _fc0_body,
        grid=(GRID,),
        in_specs=[_row_spec(), _row_spec(), _full_spec((D, D)),
                  _full_spec((D, D)), _full_spec((1, D))],
        out_specs=[_row_spec(), _full_spec((1, D)), _full_spec((1, D))],
        out_shape=[
            jax.ShapeDtypeStruct((N, D), jnp.float32),
            jax.ShapeDtypeStruct((1, D), jnp.float32),
            jax.ShapeDtypeStruct((1, D), jnp.float32),
        ],
    )(x, agg, wa, wb, b)


def _bn_norm_pool(z, s, ss, gamma, beta, wp, bp):
    return pl.pallas_call(
        _bn_body,
        grid=(GRID,),
        in_specs=[_row_spec(), _full_spec((1, D)), _full_spec((1, D)),
                  _full_spec((1, D)), _full_spec((1, D)),
                  _full_spec((D, D)), _full_spec((1, D))],
        out_specs=[_row_spec(), _row_spec()],
        out_shape=[
            jax.ShapeDtypeStruct((N, D), jnp.float32),
            jax.ShapeDtypeStruct((N, D), jnp.float32),
        ],
    )(z, s, ss, gamma, beta, wp, bp)


def _fc1(x, agg, wa, wb, b):
    return pl.pallas_call(
        _fc1_body,
        grid=(GRID,),
        in_specs=[_row_spec(), _row_spec(), _full_spec((D, D)),
                  _full_spec((D, D)), _full_spec((1, D))],
        out_specs=_row_spec(),
        out_shape=jax.ShapeDtypeStruct((N, D), jnp.float32),
    )(x, agg, wa, wb, b)


@jax.jit
def _run(features, neigh_idx, W_pool0, b_pool0, W_fc0, b_fc0, bn_gamma,
         bn_beta, W_pool1, b_pool1, W_fc1, b_fc1):
    idx_flat = jnp.pad(neigh_idx.astype(jnp.int32).reshape(-1),
                       (0, (NP - N) * S))

    b_pool0 = b_pool0.reshape(1, D)
    b_fc0 = b_fc0.reshape(1, D)
    b_pool1 = b_pool1.reshape(1, D)
    b_fc1 = b_fc1.reshape(1, D)
    gamma = bn_gamma.reshape(1, D)
    beta = bn_beta.reshape(1, D)
    wa0, wb0 = W_fc0[:D], W_fc0[D:]
    wa1, wb1 = W_fc1[:D], W_fc1[D:]

    h0 = _mm_relu(features, W_pool0, b_pool0)
    agg0 = _sc_gather_max(h0, idx_flat)[:N]
    z, s, ss = _fc0(features, agg0, wa0, wb0, b_fc0)
    out1, h1 = _bn_norm_pool(z, s, ss, gamma, beta, W_pool1, b_pool1)
    agg1 = _sc_gather_max(h1, idx_flat)[:N]
    return _fc1(out1, agg1, wa1, wb1, b_fc1)


def kernel(features, neigh_idx, W_pool0, b_pool0, W_fc0, b_fc0, bn_gamma,
           bn_beta, W_pool1, b_pool1, W_fc1, b_fc1):
    return _run(features, neigh_idx, W_pool0, b_pool0, W_fc0, b_fc0,
                bn_gamma, bn_beta, W_pool1, b_pool1, W_fc1, b_fc1)
